# baseline (speedup 1.0000x reference)
import jax
import jax.numpy as jnp
from jax import lax
from jax.experimental import pallas as pl
from jax.experimental.pallas import tpu as pltpu
from jax.experimental.pallas import tpu_sc as plsc

BATCH = 1024
NUM_FEATURES = 768
MARGIN = 0.1
TAU = 0.02
_NEG_INF = -1e30

_NC, _NS = 2, 16
_NW = _NC * _NS
_ROWS_PER_W = BATCH // _NW
_CHUNK = _ROWS_PER_W // 2


def _gather_body(idx_hbm, table_hbm, out_hbm, idx_v, rows0_v, rows1_v,
                 sem_g0, sem_g1, sem_w0, sem_w1):
    wid = lax.axis_index("s") * _NC + lax.axis_index("c")
    base = wid * _ROWS_PER_W
    pltpu.sync_copy(idx_hbm.at[pl.ds(base, _ROWS_PER_W)], idx_v)
    g0 = pltpu.async_copy(table_hbm.at[idx_v.at[pl.ds(0, _CHUNK)]], rows0_v, sem_g0)
    g1 = pltpu.async_copy(table_hbm.at[idx_v.at[pl.ds(_CHUNK, _CHUNK)]], rows1_v, sem_g1)
    g0.wait()
    w0 = pltpu.async_copy(rows0_v, out_hbm.at[pl.ds(base, _CHUNK)], sem_w0)
    g1.wait()
    w1 = pltpu.async_copy(rows1_v, out_hbm.at[pl.ds(base + _CHUNK, _CHUNK)], sem_w1)
    w0.wait()
    w1.wait()


def _sc_gather(targets, features):
    mesh = plsc.VectorSubcoreMesh(core_axis_name="c", subcore_axis_name="s")
    k = pl.kernel(
        _gather_body,
        mesh=mesh,
        out_type=jax.ShapeDtypeStruct((BATCH, NUM_FEATURES), jnp.float32),
        scratch_types=[
            pltpu.VMEM((_ROWS_PER_W,), jnp.int32),
            pltpu.VMEM((_CHUNK, NUM_FEATURES), jnp.float32),
            pltpu.VMEM((_CHUNK, NUM_FEATURES), jnp.float32),
            pltpu.SemaphoreType.DMA,
            pltpu.SemaphoreType.DMA,
            pltpu.SemaphoreType.DMA,
            pltpu.SemaphoreType.DMA,
        ],
    )
    return k(targets.astype(jnp.int32), features)


_BLK = 256
_NBLK = BATCH // _BLK


def _loss_body(x_ref, cl_ref, trow_ref, tcol_ref, out_ref,
               m2_ref, d2_ref, p2_ref, neg2_ref, colsum_ref, acc_ref):
    i = pl.program_id(0)

    @pl.when(i == 0)
    def _init():
        m2_ref[...] = jnp.full((1, BATCH), _NEG_INF, jnp.float32)
        d2_ref[...] = jnp.zeros((1, BATCH), jnp.float32)
        p2_ref[...] = jnp.zeros((1, BATCH), jnp.float32)
        neg2_ref[...] = jnp.full((1, BATCH), _NEG_INF, jnp.float32)
        colsum_ref[...] = jnp.zeros((1, BATCH), jnp.float32)
        acc_ref[0, 0] = jnp.float32(0.0)

    x = x_ref[...]
    n = jnp.sqrt(jnp.sum(x * x, axis=1, keepdims=True))
    xi = (x / jnp.maximum(n, 1e-12)).astype(jnp.bfloat16)
    scores = lax.dot_general(
        xi, cl_ref[...].astype(jnp.bfloat16), (((1,), (1,)), ((), ())),
        preferred_element_type=jnp.float32,
    )
    labels = tcol_ref[...] == trow_ref[...]
    s_tau = scores * jnp.float32(1.0 / TAU)
    masked = jnp.where(labels, s_tau, _NEG_INF)

    m1 = jnp.max(masked, axis=1, keepdims=True)
    e1 = jnp.exp(masked - m1)
    pos1 = jnp.sum(e1 * scores, axis=1, keepdims=True) / jnp.sum(
        e1, axis=1, keepdims=True
    )
    neg1 = jnp.max(jnp.where(labels, _NEG_INF, scores), axis=1, keepdims=True)
    c1 = jnp.maximum(MARGIN + neg1 - pos1, 0.0)
    acc_ref[0, 0] += jnp.sum(c1)

    bm2 = jnp.max(masked, axis=0, keepdims=True)
    m2_new = jnp.maximum(m2_ref[...], bm2)
    scale = jnp.exp(m2_ref[...] - m2_new)
    e2 = jnp.where(labels, jnp.exp(s_tau - m2_new), 0.0)
    d2_ref[...] = d2_ref[...] * scale + jnp.sum(e2, axis=0, keepdims=True)
    p2_ref[...] = p2_ref[...] * scale + jnp.sum(e2 * scores, axis=0, keepdims=True)
    m2_ref[...] = m2_new
    neg2_ref[...] = jnp.maximum(
        neg2_ref[...],
        jnp.max(jnp.where(labels, _NEG_INF, scores), axis=0, keepdims=True),
    )
    colsum_ref[...] = colsum_ref[...] + jnp.sum(scores, axis=0, keepdims=True)

    @pl.when(i == _NBLK - 1)
    def _final():
        pos2 = p2_ref[...] / d2_ref[...]
        c2 = jnp.maximum(MARGIN + neg2_ref[...] - pos2, 0.0)
        tri = acc_ref[0, 0] + jnp.sum(c2)
        center = 1.0 - jnp.sum(colsum_ref[...]) * jnp.float32(1.0 / (BATCH * BATCH))
        out_ref[0, 0] = tri + 0.08 * center


def _tc_loss(i_feats, cl, targets):
    t = targets.astype(jnp.int32)
    out = pl.pallas_call(
        _loss_body,
        grid=(_NBLK,),
        in_specs=[
            pl.BlockSpec((_BLK, NUM_FEATURES), lambda i: (i, 0)),
            pl.BlockSpec((BATCH, NUM_FEATURES), lambda i: (0, 0)),
            pl.BlockSpec((1, BATCH), lambda i: (0, 0)),
            pl.BlockSpec((_BLK, 1), lambda i: (i, 0)),
        ],
        out_specs=pl.BlockSpec((1, 1), lambda i: (0, 0), memory_space=pltpu.SMEM),
        out_shape=jax.ShapeDtypeStruct((1, 1), jnp.float32),
        scratch_shapes=[
            pltpu.VMEM((1, BATCH), jnp.float32),
            pltpu.VMEM((1, BATCH), jnp.float32),
            pltpu.VMEM((1, BATCH), jnp.float32),
            pltpu.VMEM((1, BATCH), jnp.float32),
            pltpu.VMEM((1, BATCH), jnp.float32),
            pltpu.SMEM((1, 1), jnp.float32),
        ],
    )(i_feats, cl, t.reshape(1, BATCH), t.reshape(BATCH, 1))
    return out[0, 0]


def kernel(i_feats, targets, features):
    cl = _sc_gather(targets, features)
    return _tc_loss(i_feats, cl, targets)


# speedup vs baseline: 1.0429x; 1.0429x over previous
import jax
import jax.numpy as jnp
from jax import lax
from jax.experimental import pallas as pl
from jax.experimental.pallas import tpu as pltpu
from jax.experimental.pallas import tpu_sc as plsc

BATCH = 1024
NUM_FEATURES = 768
MARGIN = 0.1
TAU = 0.02
_NEG_INF = -1e30

_NC, _NS = 2, 16
_NW = _NC * _NS
_ROWS_PER_W = BATCH // _NW
_CHUNK = _ROWS_PER_W // 2


def _gather_body(idx_hbm, table_hbm, out_hbm, idx_v, rows0_v, rows1_v,
                 sem_g0, sem_g1, sem_w0, sem_w1):
    wid = lax.axis_index("s") * _NC + lax.axis_index("c")
    base = wid * _ROWS_PER_W
    pltpu.sync_copy(idx_hbm.at[pl.ds(base, _ROWS_PER_W)], idx_v)
    g0 = pltpu.async_copy(table_hbm.at[idx_v.at[pl.ds(0, _CHUNK)]], rows0_v, sem_g0)
    g1 = pltpu.async_copy(table_hbm.at[idx_v.at[pl.ds(_CHUNK, _CHUNK)]], rows1_v, sem_g1)
    g0.wait()
    w0 = pltpu.async_copy(rows0_v, out_hbm.at[pl.ds(base, _CHUNK)], sem_w0)
    g1.wait()
    w1 = pltpu.async_copy(rows1_v, out_hbm.at[pl.ds(base + _CHUNK, _CHUNK)], sem_w1)
    w0.wait()
    w1.wait()


def _sc_gather(targets, features):
    mesh = plsc.VectorSubcoreMesh(core_axis_name="c", subcore_axis_name="s")
    k = pl.kernel(
        _gather_body,
        mesh=mesh,
        out_type=jax.ShapeDtypeStruct((BATCH, NUM_FEATURES), jnp.float32),
        scratch_types=[
            pltpu.VMEM((_ROWS_PER_W,), jnp.int32),
            pltpu.VMEM((_CHUNK, NUM_FEATURES), jnp.float32),
            pltpu.VMEM((_CHUNK, NUM_FEATURES), jnp.float32),
            pltpu.SemaphoreType.DMA,
            pltpu.SemaphoreType.DMA,
            pltpu.SemaphoreType.DMA,
            pltpu.SemaphoreType.DMA,
        ],
    )
    return k(targets.astype(jnp.int32), features)


_BLK = 256
_NBLK = BATCH // _BLK


def _loss_body(x_ref, cl_ref, trow_ref, tcol_ref, out_ref,
               d2_ref, p2_ref, neg2_ref, xsum_ref, acc_ref):
    i = pl.program_id(0)

    @pl.when(i == 0)
    def _init():
        d2_ref[...] = jnp.zeros((1, BATCH), jnp.float32)
        p2_ref[...] = jnp.zeros((1, BATCH), jnp.float32)
        neg2_ref[...] = jnp.full((1, BATCH), _NEG_INF, jnp.float32)
        xsum_ref[...] = jnp.zeros((1, NUM_FEATURES), jnp.float32)
        acc_ref[0, 0] = jnp.float32(0.0)

    x = x_ref[...]
    n = jnp.sqrt(jnp.sum(x * x, axis=1, keepdims=True))
    xi = x / jnp.maximum(n, 1e-12)
    xsum_ref[...] += jnp.sum(xi, axis=0, keepdims=True)
    st = lax.dot_general(
        (xi * jnp.float32(1.0 / TAU)).astype(jnp.bfloat16),
        cl_ref[...].astype(jnp.bfloat16), (((1,), (1,)), ((), ())),
        preferred_element_type=jnp.float32,
    )
    labels = tcol_ref[...] == trow_ref[...]
    E = jnp.where(labels, jnp.exp(st), 0.0)
    ES = E * st
    nm = jnp.where(labels, _NEG_INF, st)

    pos1 = jnp.sum(ES, axis=1, keepdims=True) / jnp.sum(E, axis=1, keepdims=True)
    neg1 = jnp.max(nm, axis=1, keepdims=True)
    c1 = jnp.maximum(MARGIN + jnp.float32(TAU) * (neg1 - pos1), 0.0)
    acc_ref[0, 0] += jnp.sum(c1)

    d2_ref[...] += jnp.sum(E, axis=0, keepdims=True)
    p2_ref[...] += jnp.sum(ES, axis=0, keepdims=True)
    neg2_ref[...] = jnp.maximum(neg2_ref[...], jnp.max(nm, axis=0, keepdims=True))

    @pl.when(i == _NBLK - 1)
    def _final():
        pos2 = p2_ref[...] / d2_ref[...]
        c2 = jnp.maximum(MARGIN + jnp.float32(TAU) * (neg2_ref[...] - pos2), 0.0)
        tri = acc_ref[0, 0] + jnp.sum(c2)
        clsum = jnp.sum(cl_ref[...], axis=0, keepdims=True)
        total = jnp.sum(xsum_ref[...] * clsum)
        center = 1.0 - total * jnp.float32(1.0 / (BATCH * BATCH))
        out_ref[0, 0] = tri + 0.08 * center


def _tc_loss(i_feats, cl, targets):
    t = targets.astype(jnp.int32)
    out = pl.pallas_call(
        _loss_body,
        grid=(_NBLK,),
        in_specs=[
            pl.BlockSpec((_BLK, NUM_FEATURES), lambda i: (i, 0)),
            pl.BlockSpec((BATCH, NUM_FEATURES), lambda i: (0, 0)),
            pl.BlockSpec((1, BATCH), lambda i: (0, 0)),
            pl.BlockSpec((_BLK, 1), lambda i: (i, 0)),
        ],
        out_specs=pl.BlockSpec((1, 1), lambda i: (0, 0), memory_space=pltpu.SMEM),
        out_shape=jax.ShapeDtypeStruct((1, 1), jnp.float32),
        scratch_shapes=[
            pltpu.VMEM((1, BATCH), jnp.float32),
            pltpu.VMEM((1, BATCH), jnp.float32),
            pltpu.VMEM((1, BATCH), jnp.float32),
            pltpu.VMEM((1, NUM_FEATURES), jnp.float32),
            pltpu.SMEM((1, 1), jnp.float32),
        ],
    )(i_feats, cl, t.reshape(1, BATCH), t.reshape(BATCH, 1))
    return out[0, 0]


def kernel(i_feats, targets, features):
    cl = _sc_gather(targets, features)
    return _tc_loss(i_feats, cl, targets)


# speedup vs baseline: 1.0726x; 1.0285x over previous
import jax
import jax.numpy as jnp
from jax import lax
from jax.experimental import pallas as pl
from jax.experimental.pallas import tpu as pltpu
from jax.experimental.pallas import tpu_sc as plsc

BATCH = 1024
NUM_FEATURES = 768
MARGIN = 0.1
TAU = 0.02
_NEG_INF = -1e30

_NC, _NS = 2, 16
_NW = _NC * _NS
_ROWS_PER_W = BATCH // _NW
_CHUNK = _ROWS_PER_W // 2


def _gather_body(idx_hbm, table_hbm, out_hbm, idx_v, rows0_v, rows1_v,
                 sem_g0, sem_g1, sem_w0, sem_w1):
    wid = lax.axis_index("s") * _NC + lax.axis_index("c")
    base = wid * _ROWS_PER_W
    pltpu.sync_copy(idx_hbm.at[pl.ds(base, _ROWS_PER_W)], idx_v)
    g0 = pltpu.async_copy(table_hbm.at[idx_v.at[pl.ds(0, _CHUNK)]], rows0_v, sem_g0)
    g1 = pltpu.async_copy(table_hbm.at[idx_v.at[pl.ds(_CHUNK, _CHUNK)]], rows1_v, sem_g1)
    g0.wait()
    w0 = pltpu.async_copy(rows0_v, out_hbm.at[pl.ds(base, _CHUNK)], sem_w0)
    g1.wait()
    w1 = pltpu.async_copy(rows1_v, out_hbm.at[pl.ds(base + _CHUNK, _CHUNK)], sem_w1)
    w0.wait()
    w1.wait()


def _sc_gather(targets, features):
    mesh = plsc.VectorSubcoreMesh(core_axis_name="c", subcore_axis_name="s")
    k = pl.kernel(
        _gather_body,
        mesh=mesh,
        out_type=jax.ShapeDtypeStruct((BATCH, NUM_FEATURES), jnp.float32),
        scratch_types=[
            pltpu.VMEM((_ROWS_PER_W,), jnp.int32),
            pltpu.VMEM((_CHUNK, NUM_FEATURES), jnp.float32),
            pltpu.VMEM((_CHUNK, NUM_FEATURES), jnp.float32),
            pltpu.SemaphoreType.DMA,
            pltpu.SemaphoreType.DMA,
            pltpu.SemaphoreType.DMA,
            pltpu.SemaphoreType.DMA,
        ],
    )
    return k(targets.astype(jnp.int32), features)


_BLK = 256
_NBLK = BATCH // _BLK


def _norm_body(x_ref, xi_ref):
    x = x_ref[...]
    n = jnp.sqrt(jnp.sum(x * x, axis=1, keepdims=True))
    xi_ref[...] = (
        x * (jnp.float32(1.0 / TAU) / jnp.maximum(n, 1e-12))
    ).astype(jnp.bfloat16)


def _tc_norm(i_feats):
    return pl.pallas_call(
        _norm_body,
        grid=(_NBLK,),
        in_specs=[pl.BlockSpec((_BLK, NUM_FEATURES), lambda i: (i, 0))],
        out_specs=pl.BlockSpec((_BLK, NUM_FEATURES), lambda i: (i, 0)),
        out_shape=jax.ShapeDtypeStruct((BATCH, NUM_FEATURES), jnp.bfloat16),
    )(i_feats)


def _loss_body(xi_ref, cl_ref, trow_ref, tcolb_ref, out_ref,
               sumE_ref, sumES_ref, negr_ref, acc_ref, ssum_ref):
    j = pl.program_id(0)

    @pl.when(j == 0)
    def _init():
        sumE_ref[...] = jnp.zeros((BATCH, 1), jnp.float32)
        sumES_ref[...] = jnp.zeros((BATCH, 1), jnp.float32)
        negr_ref[...] = jnp.full((BATCH, 1), _NEG_INF, jnp.float32)
        acc_ref[0, 0] = jnp.float32(0.0)
        ssum_ref[0, 0] = jnp.float32(0.0)

    st = lax.dot_general(
        xi_ref[...], cl_ref[...].astype(jnp.bfloat16), (((1,), (1,)), ((), ())),
        preferred_element_type=jnp.float32,
    )
    labels = trow_ref[...] == tcolb_ref[...]
    E = jnp.where(labels, jnp.exp(st), 0.0)
    ES = E * st
    nm = jnp.where(labels, _NEG_INF, st)

    pos2 = jnp.sum(ES, axis=0, keepdims=True) / jnp.sum(E, axis=0, keepdims=True)
    neg2 = jnp.max(nm, axis=0, keepdims=True)
    c2 = jnp.maximum(MARGIN + jnp.float32(TAU) * (neg2 - pos2), 0.0)
    acc_ref[0, 0] += jnp.sum(c2)
    ssum_ref[0, 0] += jnp.sum(st)

    sumE_ref[...] += jnp.sum(E, axis=1, keepdims=True)
    sumES_ref[...] += jnp.sum(ES, axis=1, keepdims=True)
    negr_ref[...] = jnp.maximum(negr_ref[...], jnp.max(nm, axis=1, keepdims=True))

    @pl.when(j == _NBLK - 1)
    def _final():
        pos1 = sumES_ref[...] / sumE_ref[...]
        c1 = jnp.maximum(MARGIN + jnp.float32(TAU) * (negr_ref[...] - pos1), 0.0)
        tri = acc_ref[0, 0] + jnp.sum(c1)
        center = 1.0 - ssum_ref[0, 0] * jnp.float32(TAU / (BATCH * BATCH))
        out_ref[0, 0] = tri + 0.08 * center


def _tc_loss(xi, cl, targets):
    t = targets.astype(jnp.int32)
    out = pl.pallas_call(
        _loss_body,
        grid=(_NBLK,),
        in_specs=[
            pl.BlockSpec((BATCH, NUM_FEATURES), lambda j: (0, 0)),
            pl.BlockSpec((_BLK, NUM_FEATURES), lambda j: (j, 0)),
            pl.BlockSpec((BATCH, 1), lambda j: (0, 0)),
            pl.BlockSpec((1, _BLK), lambda j: (0, j)),
        ],
        out_specs=pl.BlockSpec((1, 1), lambda j: (0, 0), memory_space=pltpu.SMEM),
        out_shape=jax.ShapeDtypeStruct((1, 1), jnp.float32),
        scratch_shapes=[
            pltpu.VMEM((BATCH, 1), jnp.float32),
            pltpu.VMEM((BATCH, 1), jnp.float32),
            pltpu.VMEM((BATCH, 1), jnp.float32),
            pltpu.SMEM((1, 1), jnp.float32),
            pltpu.SMEM((1, 1), jnp.float32),
        ],
    )(xi, cl, t.reshape(BATCH, 1), t.reshape(1, BATCH))
    return out[0, 0]


def kernel(i_feats, targets, features):
    cl = _sc_gather(targets, features)
    xi = _tc_norm(i_feats)
    return _tc_loss(xi, cl, targets)
